# baseline (speedup 1.0000x reference)
import functools
import jax
import jax.numpy as jnp
from jax import lax
from jax.experimental import pallas as pl
from jax.experimental.pallas import tpu as pltpu
from jax.experimental.pallas import tpu_sc as plsc

NC, NS = 2, 16
NW = NC * NS
N, E = 10000, 320000
H = 256
FH = 128
HALF = N // NC
DUMP = HALF
ACC_ROWS = HALF + 8
CHUNK = 128
WB = 320
PCH = 40

_mesh = plsc.VectorSubcoreMesh(core_axis_name="c", subcore_axis_name="s")
_sc_params = pltpu.CompilerParams(needs_layout_passes=False)


@functools.partial(
    pl.kernel,
    mesh=_mesh,
    out_type=jax.ShapeDtypeStruct((NW, N), jnp.float32),
    scratch_types=[
        pltpu.VMEM((E // NW,), jnp.int32),
        pltpu.VMEM((N,), jnp.float32),
    ],
    compiler_params=_sc_params,
)
def _sc_degree(dst_hbm, out_hbm, dstv, hist):
    wid = lax.axis_index("s") * NC + lax.axis_index("c")
    epw = E // NW
    pltpu.sync_copy(dst_hbm.at[pl.ds(wid * epw, epw)], dstv)
    zeros16 = jnp.zeros((16,), jnp.float32)

    def zero_body(i, carry):
        hist[pl.ds(i * 16, 16)] = zeros16
        return carry

    lax.fori_loop(0, N // 16, zero_body, 0)
    ones16 = jnp.ones((16,), jnp.float32)

    def acc_body(i, carry):
        idx = dstv[pl.ds(i * 16, 16)]
        plsc.addupdate_scatter(hist, [idx], ones16)
        return carry

    lax.fori_loop(0, epw // 16, acc_body, 0)
    pltpu.sync_copy(hist, out_hbm.at[wid])


_EPT = E // NW
_NCMAX = (_EPT + 2 * CHUNK) // CHUNK + 1
_CAP = _NCMAX * CHUNK


@functools.partial(
    pl.kernel,
    mesh=_mesh,
    out_type=[
        jax.ShapeDtypeStruct((NC, NW, _CAP), jnp.int32),
        jax.ShapeDtypeStruct((NC, NW, _CAP), jnp.int32),
        jax.ShapeDtypeStruct((NC, NW, 16), jnp.int32),
    ],
    scratch_types=[
        pltpu.VMEM((_EPT,), jnp.int32),
        pltpu.VMEM((_EPT,), jnp.int32),
        pltpu.VMEM((_CAP,), jnp.int32),
        pltpu.VMEM((_CAP,), jnp.int32),
        pltpu.VMEM((_CAP,), jnp.int32),
        pltpu.VMEM((_CAP,), jnp.int32),
        pltpu.VMEM((16,), jnp.int32),
    ],
    compiler_params=_sc_params,
)
def _sc_partition(src_hbm, dst_hbm, srcp_hbm, dstp_hbm, cnt_hbm,
                  srcv, dstv, sa, da, sb, db, cntv):
    wid = lax.axis_index("s") * NC + lax.axis_index("c")
    base = wid * _EPT
    pltpu.sync_copy(src_hbm.at[pl.ds(base, _EPT)], srcv)
    pltpu.sync_copy(dst_hbm.at[pl.ds(base, _EPT)], dstv)

    def part_body(i, carry):
        offa, offb = carry
        s16 = srcv[pl.ds(i * 16, 16)]
        d16 = dstv[pl.ds(i * 16, 16)]
        in_a = d16 < HALF
        plsc.store_compressed(sa.at[pl.ds(offa, 16)], s16, mask=in_a)
        plsc.store_compressed(da.at[pl.ds(offa, 16)], d16, mask=in_a)
        plsc.store_compressed(sb.at[pl.ds(offb, 16)], s16, mask=~in_a)
        plsc.store_compressed(db.at[pl.ds(offb, 16)], d16 - HALF, mask=~in_a)
        ca = jnp.sum(in_a.astype(jnp.int32))
        return offa + ca, offb + (16 - ca)

    offa, offb = lax.fori_loop(0, _EPT // 16, part_body,
                               (jnp.int32(0), jnp.int32(0)))

    dump16 = jnp.full((16,), DUMP, jnp.int32)
    zero16 = jnp.zeros((16,), jnp.int32)
    for k in range(2 * CHUNK // 16):
        sa[pl.ds(offa + k * 16, 16)] = zero16
        da[pl.ds(offa + k * 16, 16)] = dump16
        sb[pl.ds(offb + k * 16, 16)] = zero16
        db[pl.ds(offb + k * 16, 16)] = dump16

    nca = (offa + 2 * CHUNK - 1) // (2 * CHUNK) * 2
    ncb = (offb + 2 * CHUNK - 1) // (2 * CHUNK) * 2
    cntv[pl.ds(0, 16)] = jnp.full((16,), nca, jnp.int32)
    pltpu.sync_copy(cntv, cnt_hbm.at[0, wid])
    cntv[pl.ds(0, 16)] = jnp.full((16,), ncb, jnp.int32)
    pltpu.sync_copy(cntv, cnt_hbm.at[1, wid])
    pltpu.sync_copy(sa, srcp_hbm.at[0, wid])
    pltpu.sync_copy(da, dstp_hbm.at[0, wid])
    pltpu.sync_copy(sb, srcp_hbm.at[1, wid])
    pltpu.sync_copy(db, dstp_hbm.at[1, wid])


_ZB = 80


@functools.partial(
    pl.kernel,
    mesh=_mesh,
    out_type=jax.ShapeDtypeStruct((N, FH), jnp.float32),
    scratch_types=[
        pltpu.VMEM((PCH * CHUNK,), jnp.int32),
        pltpu.VMEM((PCH, CHUNK), jnp.int32),
        pltpu.VMEM((16,), jnp.int32),
        pltpu.VMEM((CHUNK, FH), jnp.float32),
        pltpu.VMEM((CHUNK, FH), jnp.float32),
        pltpu.VMEM_SHARED((ACC_ROWS, FH), jnp.float32),
        pltpu.SemaphoreType.DMA,
        pltpu.SemaphoreType.DMA,
    ],
    compiler_params=_sc_params,
)
def _sc_scatter(zh_hbm, srcp_hbm, dstp_hbm, cnt_hbm, out_hbm, src1d, dst2d,
                cntv, rows0, rows1, accum, sem0, sem1):
    c = lax.axis_index("c")
    s = lax.axis_index("s")
    zeros16 = jnp.zeros((16,), jnp.float32)

    def zero_body(r, carry):
        for k in range(FH // 16):
            rows0[r, pl.ds(k * 16, 16)] = zeros16
        return carry

    lax.fori_loop(0, _ZB, zero_body, 0)
    start = pl.multiple_of(jnp.minimum(s * WB, HALF - WB), 8)
    for j in range(WB // _ZB):
        pltpu.sync_copy(rows0.at[pl.ds(0, _ZB)],
                        accum.at[pl.ds(start + j * _ZB, _ZB)])
    plsc.subcore_barrier()

    rows = (rows0, rows1)
    sems = (sem0, sem1)

    for b_ in range(2):
        b = 2 * s + b_
        pltpu.sync_copy(cnt_hbm.at[c, b], cntv)
        nc = jnp.max(cntv[pl.ds(0, 16)])
        for p in range(2):
            np_ = jnp.clip(nc - p * PCH, 0, PCH)

            @pl.when(np_ > 0)
            def _():
                pltpu.sync_copy(
                    srcp_hbm.at[c, b, pl.ds(p * PCH * CHUNK, PCH * CHUNK)],
                    src1d)
                pltpu.sync_copy(dstp_hbm.at[c, b, pl.ds(p * PCH, PCH)],
                                dst2d)
                pltpu.async_copy(zh_hbm.at[src1d.at[pl.ds(0, CHUNK)]],
                                 rows0, sem0)
                pltpu.async_copy(zh_hbm.at[src1d.at[pl.ds(CHUNK, CHUNK)]],
                                 rows1, sem1)

                def group_body(g, carry):
                    for j in range(2):
                        i = g * 2 + j
                        pltpu.make_async_copy(
                            zh_hbm.at[src1d.at[pl.ds(i * CHUNK, CHUNK)]],
                            rows[j], sems[j]).wait()
                        pltpu.sync_copy(rows[j], accum.at[dst2d.at[i]],
                                        add=True)
                        nxt = i + 2

                        @pl.when(nxt < np_)
                        def _():
                            pltpu.async_copy(
                                zh_hbm.at[
                                    src1d.at[pl.ds(nxt * CHUNK, CHUNK)]],
                                rows[j], sems[j])
                    return carry

                lax.fori_loop(0, np_ // 2, group_body, 0)

    plsc.subcore_barrier()
    pltpu.sync_copy(accum.at[pl.ds(start, WB)],
                    out_hbm.at[pl.ds(c * HALF + start, WB)])


_BLK = 1000


def _tc0_body(degp_ref, dinv_ref):
    deg = jnp.sum(degp_ref[...], axis=0, keepdims=True) + 1.0
    dinv_ref[...] = lax.rsqrt(deg)


def _tc1_body(x_ref, w_ref, dinv_ref, za_ref, zb_ref):
    dinv = dinv_ref[...]
    xw = jnp.dot(x_ref[...], w_ref[...], preferred_element_type=jnp.float32)
    z = xw * dinv
    za_ref[...] = z[:, :FH]
    zb_ref[...] = z[:, FH:]


def _mid_h(sa, sb, za, zb, b_ref, dinv):
    agg_a = sa[...] + za[...]
    agg_b = sb[...] + zb[...]
    h = jnp.concatenate([agg_a, agg_b], axis=1) * dinv + b_ref[...]
    return jnp.maximum(h, 0.0)


def _tc2_body(sa, sb, za, zb, dinv_ref, w_ref, b_ref, za2_ref, zb2_ref):
    dinv = dinv_ref[...]
    h = _mid_h(sa, sb, za, zb, b_ref, dinv)
    znew = jnp.dot(h, w_ref[...], preferred_element_type=jnp.float32)
    znew = znew * dinv
    za2_ref[...] = znew[:, :FH]
    zb2_ref[...] = znew[:, FH:]


def _tc3_body(sa, sb, za, zb, dinv_ref, b2_ref, wl_ref, bl_ref, out_ref):
    dinv = dinv_ref[...]
    h = _mid_h(sa, sb, za, zb, b2_ref, dinv)
    out_ref[...] = (
        jnp.dot(h, wl_ref[...], preferred_element_type=jnp.float32)
        + bl_ref[...]
    )


def _row_spec(width):
    return pl.BlockSpec((_BLK, width), lambda i: (i, 0))


def _full_spec(shape):
    nd = len(shape)
    return pl.BlockSpec(shape, lambda i: (0,) * nd)


def _half_shapes():
    return [jax.ShapeDtypeStruct((N, FH), jnp.float32) for _ in range(2)]


def kernel(x, edge_index, W1, b1, W2, b2, Wl, bl):
    src = edge_index[0]
    dst = edge_index[1]
    C = Wl.shape[1]
    grid = (N // _BLK,)

    degp = _sc_degree(dst)
    srcp, dstp, cnts = _sc_partition(src, dst)
    dstp = dstp.reshape(NC, NW, _NCMAX, CHUNK)

    dinv_row = pl.pallas_call(
        _tc0_body,
        in_specs=[pl.BlockSpec((NW, N), lambda: (0, 0))],
        out_specs=pl.BlockSpec((1, N), lambda: (0, 0)),
        out_shape=jax.ShapeDtypeStruct((1, N), jnp.float32),
    )(degp)
    dinv = dinv_row.reshape(N, 1)

    za, zb = pl.pallas_call(
        _tc1_body,
        grid=grid,
        in_specs=[
            _row_spec(x.shape[1]),
            _full_spec(W1.shape),
            _row_spec(1),
        ],
        out_specs=[_row_spec(FH)] * 2,
        out_shape=_half_shapes(),
    )(x, W1, dinv)

    sa = _sc_scatter(za, srcp, dstp, cnts)
    sb = _sc_scatter(zb, srcp, dstp, cnts)

    za2, zb2 = pl.pallas_call(
        _tc2_body,
        grid=grid,
        in_specs=[_row_spec(FH)] * 4
        + [_row_spec(1), _full_spec(W2.shape), _full_spec((1, H))],
        out_specs=[_row_spec(FH)] * 2,
        out_shape=_half_shapes(),
    )(sa, sb, za, zb, dinv, W2, b1.reshape(1, H))

    sa2 = _sc_scatter(za2, srcp, dstp, cnts)
    sb2 = _sc_scatter(zb2, srcp, dstp, cnts)

    out = pl.pallas_call(
        _tc3_body,
        grid=grid,
        in_specs=[_row_spec(FH)] * 4
        + [_row_spec(1), _full_spec((1, H)), _full_spec(Wl.shape),
           _full_spec((1, C))],
        out_specs=_row_spec(C),
        out_shape=jax.ShapeDtypeStruct((N, C), jnp.float32),
    )(sa2, sb2, za2, zb2, dinv, b2.reshape(1, H), Wl, bl.reshape(1, C))

    return out


# speedup vs baseline: 1.4773x; 1.4773x over previous
import functools
import jax
import jax.numpy as jnp
from jax import lax
from jax.experimental import pallas as pl
from jax.experimental.pallas import tpu as pltpu
from jax.experimental.pallas import tpu_sc as plsc

NC, NS = 2, 16
NW = NC * NS
N, E = 10000, 320000
H = 256
FH = 128
HALF = N // NC
DUMP = HALF
ACC_ROWS = HALF + 8
CHUNK = 80
WB = 320

_mesh = plsc.VectorSubcoreMesh(core_axis_name="c", subcore_axis_name="s")
_sc_params = pltpu.CompilerParams(needs_layout_passes=False)


@functools.partial(
    pl.kernel,
    mesh=_mesh,
    out_type=jax.ShapeDtypeStruct((NW, N), jnp.float32),
    scratch_types=[
        pltpu.VMEM((E // NW,), jnp.int32),
        pltpu.VMEM((N,), jnp.float32),
    ],
    compiler_params=_sc_params,
)
def _sc_degree(dst_hbm, out_hbm, dstv, hist):
    wid = lax.axis_index("s") * NC + lax.axis_index("c")
    epw = E // NW
    pltpu.sync_copy(dst_hbm.at[pl.ds(wid * epw, epw)], dstv)
    zeros16 = jnp.zeros((16,), jnp.float32)

    def zero_body(i, carry):
        hist[pl.ds(i * 16, 16)] = zeros16
        return carry

    lax.fori_loop(0, N // 16, zero_body, 0)
    ones16 = jnp.ones((16,), jnp.float32)

    def acc_body(i, carry):
        idx = dstv[pl.ds(i * 16, 16)]
        plsc.addupdate_scatter(hist, [idx], ones16)
        return carry

    lax.fori_loop(0, epw // 16, acc_body, 0)
    pltpu.sync_copy(hist, out_hbm.at[wid])


_EPW = E // NS
_P = 5
_EPP = _EPW // _P
_NCP = _EPP // CHUNK
_ZB = 80


@functools.partial(
    pl.kernel,
    mesh=_mesh,
    out_type=jax.ShapeDtypeStruct((N, FH), jnp.float32),
    scratch_types=[
        pltpu.VMEM((_EPP,), jnp.int32),
        pltpu.VMEM((_EPP,), jnp.int32),
        pltpu.VMEM((_NCP, CHUNK), jnp.int32),
        pltpu.VMEM((CHUNK, FH), jnp.float32),
        pltpu.VMEM((CHUNK, FH), jnp.float32),
        pltpu.VMEM((_ZB, FH), jnp.float32),
        pltpu.VMEM_SHARED((ACC_ROWS, FH), jnp.float32),
        pltpu.SemaphoreType.DMA,
        pltpu.SemaphoreType.DMA,
    ],
    compiler_params=_sc_params,
)
def _sc_scatter(zh_hbm, src_hbm, dst_hbm, out_hbm, src1d, dst1d, dst2d,
                rows0, rows1, zbuf, accum, sem0, sem1):
    c = lax.axis_index("c")
    s = lax.axis_index("s")
    zeros16 = jnp.zeros((16,), jnp.float32)

    def zero_body(r, carry):
        for k in range(FH // 16):
            zbuf[r, pl.ds(k * 16, 16)] = zeros16
        return carry

    lax.fori_loop(0, _ZB, zero_body, 0)
    start = pl.multiple_of(jnp.minimum(s * WB, HALF - WB), 8)
    for j in range(WB // _ZB):
        pltpu.sync_copy(zbuf, accum.at[pl.ds(start + j * _ZB, _ZB)])
    plsc.subcore_barrier()

    cbase = c * HALF
    rows = (rows0, rows1)
    sems = (sem0, sem1)

    for p in range(_P):
        base = s * _EPW + p * _EPP
        pltpu.sync_copy(src_hbm.at[pl.ds(base, _EPP)], src1d)
        pltpu.sync_copy(dst_hbm.at[pl.ds(base, _EPP)], dst1d)

        def remap_body(i, carry):
            for k in range(CHUNK // 16):
                d16 = dst1d[pl.ds(i * CHUNK + k * 16, 16)]
                rel = d16 - cbase
                oob = (rel < 0) | (rel >= HALF)
                dst2d[i, pl.ds(k * 16, 16)] = jnp.where(oob, DUMP, rel)
            return carry

        lax.fori_loop(0, _NCP, remap_body, 0)

        pltpu.async_copy(zh_hbm.at[src1d.at[pl.ds(0, CHUNK)]], rows0, sem0)
        pltpu.async_copy(zh_hbm.at[src1d.at[pl.ds(CHUNK, CHUNK)]], rows1,
                         sem1)

        def group_body(g, carry):
            for j in range(2):
                i = g * 2 + j
                pltpu.make_async_copy(
                    zh_hbm.at[src1d.at[pl.ds(i * CHUNK, CHUNK)]], rows[j],
                    sems[j]).wait()
                pltpu.sync_copy(rows[j], accum.at[dst2d.at[i]], add=True)
                nxt = i + 2

                @pl.when(nxt < _NCP)
                def _():
                    pltpu.async_copy(
                        zh_hbm.at[src1d.at[pl.ds(nxt * CHUNK, CHUNK)]],
                        rows[j], sems[j])
            return carry

        lax.fori_loop(0, _NCP // 2, group_body, 0)

    plsc.subcore_barrier()
    pltpu.sync_copy(accum.at[pl.ds(start, WB)],
                    out_hbm.at[pl.ds(cbase + start, WB)])


_BLK = 1000


def _tc0_body(degp_ref, dinv_ref):
    deg = jnp.sum(degp_ref[...], axis=0, keepdims=True) + 1.0
    dinv_ref[...] = lax.rsqrt(deg)


def _tc1_body(x_ref, w_ref, dinv_ref, za_ref, zb_ref):
    dinv = dinv_ref[...]
    xw = jnp.dot(x_ref[...], w_ref[...], preferred_element_type=jnp.float32)
    z = xw * dinv
    za_ref[...] = z[:, :FH]
    zb_ref[...] = z[:, FH:]


def _mid_h(sa, sb, za, zb, b_ref, dinv):
    agg_a = sa[...] + za[...]
    agg_b = sb[...] + zb[...]
    h = jnp.concatenate([agg_a, agg_b], axis=1) * dinv + b_ref[...]
    return jnp.maximum(h, 0.0)


def _tc2_body(sa, sb, za, zb, dinv_ref, w_ref, b_ref, za2_ref, zb2_ref):
    dinv = dinv_ref[...]
    h = _mid_h(sa, sb, za, zb, b_ref, dinv)
    znew = jnp.dot(h, w_ref[...], preferred_element_type=jnp.float32)
    znew = znew * dinv
    za2_ref[...] = znew[:, :FH]
    zb2_ref[...] = znew[:, FH:]


def _tc3_body(sa, sb, za, zb, dinv_ref, b2_ref, wl_ref, bl_ref, out_ref):
    dinv = dinv_ref[...]
    h = _mid_h(sa, sb, za, zb, b2_ref, dinv)
    out_ref[...] = (
        jnp.dot(h, wl_ref[...], preferred_element_type=jnp.float32)
        + bl_ref[...]
    )


def _row_spec(width):
    return pl.BlockSpec((_BLK, width), lambda i: (i, 0))


def _full_spec(shape):
    nd = len(shape)
    return pl.BlockSpec(shape, lambda i: (0,) * nd)


def _half_shapes():
    return [jax.ShapeDtypeStruct((N, FH), jnp.float32) for _ in range(2)]


def kernel(x, edge_index, W1, b1, W2, b2, Wl, bl):
    src = edge_index[0]
    dst = edge_index[1]
    C = Wl.shape[1]
    grid = (N // _BLK,)

    degp = _sc_degree(dst)

    dinv_row = pl.pallas_call(
        _tc0_body,
        in_specs=[pl.BlockSpec((NW, N), lambda: (0, 0))],
        out_specs=pl.BlockSpec((1, N), lambda: (0, 0)),
        out_shape=jax.ShapeDtypeStruct((1, N), jnp.float32),
    )(degp)
    dinv = dinv_row.reshape(N, 1)

    za, zb = pl.pallas_call(
        _tc1_body,
        grid=grid,
        in_specs=[
            _row_spec(x.shape[1]),
            _full_spec(W1.shape),
            _row_spec(1),
        ],
        out_specs=[_row_spec(FH)] * 2,
        out_shape=_half_shapes(),
    )(x, W1, dinv)

    sa = _sc_scatter(za, src, dst)
    sb = _sc_scatter(zb, src, dst)

    za2, zb2 = pl.pallas_call(
        _tc2_body,
        grid=grid,
        in_specs=[_row_spec(FH)] * 4
        + [_row_spec(1), _full_spec(W2.shape), _full_spec((1, H))],
        out_specs=[_row_spec(FH)] * 2,
        out_shape=_half_shapes(),
    )(sa, sb, za, zb, dinv, W2, b1.reshape(1, H))

    sa2 = _sc_scatter(za2, src, dst)
    sb2 = _sc_scatter(zb2, src, dst)

    out = pl.pallas_call(
        _tc3_body,
        grid=grid,
        in_specs=[_row_spec(FH)] * 4
        + [_row_spec(1), _full_spec((1, H)), _full_spec(Wl.shape),
           _full_spec((1, C))],
        out_specs=_row_spec(C),
        out_shape=jax.ShapeDtypeStruct((N, C), jnp.float32),
    )(sa2, sb2, za2, zb2, dinv, b2.reshape(1, H), Wl, bl.reshape(1, C))

    return out
